# baseline (speedup 1.0000x reference)
import functools

import jax
import jax.numpy as jnp
from jax import lax
from jax.experimental import pallas as pl
from jax.experimental.pallas import tpu as pltpu
from jax.experimental.pallas import tpu_sc as plsc

_LANES = 128
_CK = 64
_NW = 32
_NCH = 4
_NRING = 8


def _make_sc_segment_sum(n_nodes, d, e_chunks, acc_rows):
    mesh = plsc.VectorSubcoreMesh(core_axis_name="c", subcore_axis_name="s")
    zero_copies = acc_rows // (16 * _LANES)
    out_rows_per_tile = acc_rows // 16
    q, rmd = divmod(e_chunks, _NW)
    assert q >= 2 * _NRING

    @functools.partial(
        pl.kernel,
        out_type=jax.ShapeDtypeStruct((2, acc_rows, d), jnp.float32),
        mesh=mesh,
        scratch_types=[pltpu.VMEM((_NRING, _CK), jnp.int32) for _ in range(2)]
        + [pltpu.VMEM((_CK, d), jnp.float32) for _ in range(_NCH)]
        + [pltpu.VMEM_SHARED((acc_rows, d), jnp.float32)]
        + [pltpu.SemaphoreType.DMA for _ in range(2 * _NRING + 2 * _NCH)],
    )
    def seg_sum(x_hbm, src_hbm, dst_hbm, out_hbm, *rest):
        sring, dring = rest[0], rest[1]
        rbuf = rest[2:2 + _NCH]
        acc = rest[2 + _NCH]
        o = 3 + _NCH
        isems = rest[o:o + _NRING]
        isemd = rest[o + _NRING:o + 2 * _NRING]
        gsem = rest[o + 2 * _NRING:o + 2 * _NRING + _NCH]
        ssem = rest[o + 2 * _NRING + _NCH:]
        c = lax.axis_index("c")
        s = lax.axis_index("s")
        w = c * 16 + s

        with jax.named_scope("zero"):
            def zrow(i, carry):
                for k in range(d // 16):
                    rbuf[0][i, pl.ds(k * 16, 16)] = jnp.zeros((16,), jnp.float32)
                return carry

            lax.fori_loop(0, _CK, zrow, 0)
            for k in range(zero_copies * 2):
                pltpu.sync_copy(
                    rbuf[0], acc.at[pl.ds((s * zero_copies * 2 + k) * _CK, _CK)])
            plsc.subcore_barrier()

        def pipeline(base, nrows):
            def step(k, j, n_static):
                t, p = k % _NRING, k % _NCH
                pltpu.make_async_copy(
                    x_hbm.at[sring.at[t]], rbuf[p], gsem[p]).wait()
                pltpu.async_copy(rbuf[p], acc.at[dring.at[t]], ssem[p], add=True)
                pltpu.make_async_copy(
                    rbuf[p], acc.at[dring.at[t]], ssem[p]).wait()
                if n_static is None or j + _NRING < n_static:
                    nt = (k + _NRING) % _NRING
                    sl = pl.ds((base + j + _NRING) * _CK, _CK)
                    pltpu.async_copy(src_hbm.at[sl], sring.at[nt], isems[nt])
                    pltpu.async_copy(dst_hbm.at[sl], dring.at[nt], isemd[nt])
                if n_static is None or j + _NCH < n_static:
                    t2 = (k + _NCH) % _NRING
                    sl2 = pl.ds((base + j + _NCH) * _CK, _CK)
                    pltpu.make_async_copy(
                        src_hbm.at[sl2], sring.at[t2], isems[t2]).wait()
                    pltpu.make_async_copy(
                        dst_hbm.at[sl2], dring.at[t2], isemd[t2]).wait()
                    p2 = (k + _NCH) % _NCH
                    pltpu.async_copy(x_hbm.at[sring.at[t2]], rbuf[p2], gsem[p2])

            for t in range(_NRING):
                sl = pl.ds((base + t) * _CK, _CK)
                pltpu.async_copy(src_hbm.at[sl], sring.at[t], isems[t])
                pltpu.async_copy(dst_hbm.at[sl], dring.at[t], isemd[t])
            for t in range(_NCH):
                sl = pl.ds((base + t) * _CK, _CK)
                pltpu.make_async_copy(src_hbm.at[sl], sring.at[t], isems[t]).wait()
                pltpu.make_async_copy(dst_hbm.at[sl], dring.at[t], isemd[t]).wait()
                pltpu.async_copy(x_hbm.at[sring.at[t]], rbuf[t], gsem[t])

            gs = (nrows - _NRING) // _NRING if nrows >= _NRING else 0

            def gbody(g, carry):
                for k in range(_NRING):
                    step(k, g * _NRING + k, None)
                return carry

            lax.fori_loop(0, gs, gbody, 0)
            for j in range(gs * _NRING, nrows):
                step(j % _NRING, j, nrows)

        with jax.named_scope("pipe"):
            if rmd:
                @pl.when(w < rmd)
                def _():
                    pipeline(w * (q + 1), q + 1)

                @pl.when(w >= rmd)
                def _():
                    pipeline(w * q + rmd, q)
            else:
                pipeline(w * q, q)

        with jax.named_scope("pipe"):
            if rmd:
                @pl.when(w < rmd)
                def _():
                    pipeline(w * (q + 1), q + 1)

                @pl.when(w >= rmd)
                def _():
                    pipeline(w * q + rmd, q)
            else:
                pipeline(w * q, q)

        with jax.named_scope("bar"):
            plsc.subcore_barrier()

        with jax.named_scope("dump"):
            rbase = s * out_rows_per_tile
            pltpu.sync_copy(
                acc.at[pl.ds(rbase, out_rows_per_tile)],
                out_hbm.at[c, pl.ds(rbase, out_rows_per_tile)],
            )

    return seg_sum


def _tc_combine(aggp, x, w_rel, b2d, w_root, relu):
    n, d = x.shape
    blk = 1000

    def body(a_ref, x_ref, wr_ref, wt_ref, b_ref, o_ref):
        a = a_ref[0] + a_ref[1]
        acc = lax.dot_general(a, wr_ref[...], (((1,), (1,)), ((), ())),
                              preferred_element_type=jnp.float32)
        acc = acc + lax.dot_general(x_ref[...], wt_ref[...], (((1,), (1,)), ((), ())),
                                    preferred_element_type=jnp.float32)
        acc = acc + b_ref[...]
        if relu:
            acc = jnp.maximum(acc, 0.0)
        o_ref[...] = acc

    return pl.pallas_call(
        body,
        grid=(n // blk,),
        in_specs=[
            pl.BlockSpec((2, blk, d), lambda i: (0, i, 0)),
            pl.BlockSpec((blk, d), lambda i: (i, 0)),
            pl.BlockSpec((d, d), lambda i: (0, 0)),
            pl.BlockSpec((d, d), lambda i: (0, 0)),
            pl.BlockSpec((1, d), lambda i: (0, 0)),
        ],
        out_specs=pl.BlockSpec((blk, d), lambda i: (i, 0)),
        out_shape=jax.ShapeDtypeStruct((n, d), jnp.float32),
    )(aggp, x, w_rel, w_root, b2d)


def kernel(x, edge_index, W1_rel, b1, W1_root, W2_rel, b2, W2_root):
    n, d = x.shape
    e = edge_index.shape[1]
    assert e % _CK == 0
    e_chunks = e // _CK
    acc_rows = -(-n // (16 * _LANES)) * (16 * _LANES)

    src = edge_index[0].astype(jnp.int32)
    dst = edge_index[1].astype(jnp.int32)

    seg_sum = _make_sc_segment_sum(n, d, e_chunks, acc_rows)
    b1_2d = b1.reshape(1, d)
    b2_2d = b2.reshape(1, d)

    aggp1 = seg_sum(x, src, dst)
    h = _tc_combine(aggp1, x, W1_rel, b1_2d, W1_root, relu=True)
    aggp2 = seg_sum(h, src, dst)
    return _tc_combine(aggp2, h, W2_rel, b2_2d, W2_root, relu=False)


# speedup vs baseline: 1.5383x; 1.5383x over previous
import functools

import jax
import jax.numpy as jnp
from jax import lax
from jax.experimental import pallas as pl
from jax.experimental.pallas import tpu as pltpu
from jax.experimental.pallas import tpu_sc as plsc

_LANES = 128
_NW = 32
_NRING = 4


def _make_sc_segment_sum(n_nodes, d, e_rows, acc_rows):
    mesh = plsc.VectorSubcoreMesh(core_axis_name="c", subcore_axis_name="s")
    zero_copies = acc_rows // (16 * _LANES)
    out_rows_per_tile = acc_rows // 16
    q, rmd = divmod(e_rows, _NW)
    assert q >= _NRING

    @functools.partial(
        pl.kernel,
        out_type=jax.ShapeDtypeStruct((2, acc_rows, d), jnp.float32),
        mesh=mesh,
        scratch_types=[pltpu.VMEM((_NRING, 2, _LANES), jnp.int32)]
        + [pltpu.VMEM((_LANES, d), jnp.float32) for _ in range(2)]
        + [pltpu.VMEM_SHARED((acc_rows, d), jnp.float32)]
        + [pltpu.SemaphoreType.DMA for _ in range(_NRING + 4)],
    )
    def seg_sum(x_hbm, edge_hbm, out_hbm, *rest):
        ering = rest[0]
        rbuf = rest[1:3]
        acc = rest[3]
        isems = rest[4:4 + _NRING]
        gsem = rest[4 + _NRING:6 + _NRING]
        ssem = rest[6 + _NRING:]
        c = lax.axis_index("c")
        s = lax.axis_index("s")
        w = c * 16 + s

        with jax.named_scope("zero"):
            def zrow(i, carry):
                for k in range(d // 16):
                    rbuf[0][i, pl.ds(k * 16, 16)] = jnp.zeros((16,), jnp.float32)
                return carry

            lax.fori_loop(0, _LANES, zrow, 0)
            for k in range(zero_copies):
                pltpu.sync_copy(
                    rbuf[0], acc.at[pl.ds((s * zero_copies + k) * _LANES, _LANES)])
            plsc.subcore_barrier()

        def pipeline(base, nrows):
            def idx_start(j):
                t = j % _NRING
                sl = pl.ds((base + j) * _LANES, _LANES)
                pltpu.async_copy(edge_hbm.at[:, sl], ering.at[t], isems[t])

            def idx_wait(j):
                t = j % _NRING
                sl = pl.ds((base + j) * _LANES, _LANES)
                pltpu.make_async_copy(edge_hbm.at[:, sl], ering.at[t], isems[t]).wait()

            def start_gather(j):
                p, t = j & 1, j % _NRING
                pltpu.async_copy(x_hbm.at[ering.at[t, 0]], rbuf[p], gsem[p])

            def wait_gather(j):
                p, t = j & 1, j % _NRING
                pltpu.make_async_copy(x_hbm.at[ering.at[t, 0]], rbuf[p], gsem[p]).wait()

            def start_scatter(j):
                p, t = j & 1, j % _NRING
                pltpu.async_copy(rbuf[p], acc.at[ering.at[t, 1]], ssem[p], add=True)

            def wait_scatter(j):
                p, t = j & 1, j % _NRING
                pltpu.make_async_copy(rbuf[p], acc.at[ering.at[t, 1]], ssem[p]).wait()

            for t in range(_NRING):
                idx_start(t)
            idx_wait(0)
            start_gather(0)
            idx_wait(1)
            start_gather(1)

            for j in range(nrows):
                wait_gather(j)
                start_scatter(j)
                wait_scatter(j)
                if j + _NRING < nrows:
                    idx_start(j + _NRING)
                if j + 2 < nrows:
                    idx_wait(j + 2)
                    start_gather(j + 2)

        with jax.named_scope("pipe"):
            if rmd:
                @pl.when(w < rmd)
                def _():
                    pipeline(w * (q + 1), q + 1)

                @pl.when(w >= rmd)
                def _():
                    pipeline(w * q + rmd, q)
            else:
                pipeline(w * q, q)

        with jax.named_scope("bar"):
            plsc.subcore_barrier()

        with jax.named_scope("dump"):
            rbase = s * out_rows_per_tile
            pltpu.sync_copy(
                acc.at[pl.ds(rbase, out_rows_per_tile)],
                out_hbm.at[c, pl.ds(rbase, out_rows_per_tile)],
            )

    return seg_sum


def _tc_combine(aggp, x, w_rel, b2d, w_root, relu):
    n, d = x.shape
    blk = 1000

    def body(a_ref, x_ref, wr_ref, wt_ref, b_ref, o_ref):
        a = a_ref[0] + a_ref[1]
        acc = lax.dot_general(a, wr_ref[...], (((1,), (1,)), ((), ())),
                              preferred_element_type=jnp.float32)
        acc = acc + lax.dot_general(x_ref[...], wt_ref[...], (((1,), (1,)), ((), ())),
                                    preferred_element_type=jnp.float32)
        acc = acc + b_ref[...]
        if relu:
            acc = jnp.maximum(acc, 0.0)
        o_ref[...] = acc

    return pl.pallas_call(
        body,
        grid=(n // blk,),
        in_specs=[
            pl.BlockSpec((2, blk, d), lambda i: (0, i, 0)),
            pl.BlockSpec((blk, d), lambda i: (i, 0)),
            pl.BlockSpec((d, d), lambda i: (0, 0)),
            pl.BlockSpec((d, d), lambda i: (0, 0)),
            pl.BlockSpec((1, d), lambda i: (0, 0)),
        ],
        out_specs=pl.BlockSpec((blk, d), lambda i: (i, 0)),
        out_shape=jax.ShapeDtypeStruct((n, d), jnp.float32),
    )(aggp, x, w_rel, w_root, b2d)


def kernel(x, edge_index, W1_rel, b1, W1_root, W2_rel, b2, W2_root):
    n, d = x.shape
    e = edge_index.shape[1]
    assert e % _LANES == 0
    e_rows = e // _LANES
    acc_rows = -(-n // (16 * _LANES)) * (16 * _LANES)

    edges = edge_index.astype(jnp.int32)

    seg_sum = _make_sc_segment_sum(n, d, e_rows, acc_rows)
    b1_2d = b1.reshape(1, d)
    b2_2d = b2.reshape(1, d)

    aggp1 = seg_sum(x, edges)
    h = _tc_combine(aggp1, x, W1_rel, b1_2d, W1_root, relu=True)
    aggp2 = seg_sum(h, edges)
    return _tc_combine(aggp2, h, W2_rel, b2_2d, W2_root, relu=False)
